# Initial kernel scaffold; baseline (speedup 1.0000x reference)
import jax
import jax.numpy as jnp
from jax.experimental import pallas as pl


def kernel(embeddings, masks_onehot):
    raise NotImplementedError("write your pallas kernel here")


# speedup vs baseline: 9.4982x; 9.4982x over previous
import numpy as np
import jax
import jax.numpy as jnp
from jax import lax
from jax.experimental import pallas as pl
from jax.experimental.pallas import tpu as pltpu

_MT_STATE = np.random.RandomState(0).get_state()
_MT_KEY0 = np.asarray(_MT_STATE[1], dtype=np.uint32)
_MT_POS0 = int(_MT_STATE[2])

_TEMP = 0.2
_K = 8
_NEG_INF = -1e30
_C0 = float(np.log1p(np.exp(-1.0)))


def _tw(mt):
    u = jnp.uint32(0x80000000)
    lo = jnp.uint32(0x7FFFFFFF)
    a = jnp.uint32(0x9908B0DF)

    def f(yv):
        return (yv >> 1) ^ jnp.where((yv & jnp.uint32(1)) != 0, a, jnp.uint32(0))

    y = (mt[:623] & u) | (mt[1:] & lo)
    n0 = mt[397:] ^ f(y[:227])
    n1 = n0 ^ f(y[227:454])
    n2 = n1[:169] ^ f(y[454:623])
    y_last = (mt[623] & u) | (n0[0] & lo)
    n_last = n1[169] ^ f(y_last)
    return jnp.concatenate([n0, n1, n2, n_last[None]])


def _nx32(state):
    mt, pos = state
    mt, pos = lax.cond(pos >= 624, lambda s: (_tw(s[0]), jnp.int32(0)), lambda s: s, (mt, pos))
    v = mt[pos]
    v = v ^ (v >> 11)
    v = v ^ ((v << 7) & jnp.uint32(0x9D2C5680))
    v = v ^ ((v << 15) & jnp.uint32(0xEFC60000))
    v = v ^ (v >> 18)
    return (mt, pos + jnp.int32(1)), v


def _rint(state, n):
    rng = (n - 1).astype(jnp.uint32)
    m = rng
    for s in (1, 2, 4, 8, 16):
        m = m | (m >> s)

    def draw(st):
        st, v = _nx32(st)
        return st, v & m

    def sample(st):
        st, v = draw(st)
        st, v = lax.while_loop(lambda c: c[1] > rng, lambda c: draw(c[0]), (st, v))
        return st, v

    return lax.cond(rng == jnp.uint32(0), lambda st: (st, jnp.uint32(0)), sample, state)


def _draw_targets(counts):
    st = (jnp.asarray(_MT_KEY0), jnp.int32(_MT_POS0))
    tr = [[None] * _K for _ in range(8)]
    vr = [[None] * _K for _ in range(4)]
    for p in range(4):
        for c in range(_K):
            n1 = counts[p, c]
            n2 = counts[p + 4, c]
            valid = (n1 > 0) & (n2 > 0)

            def do(s, n1=n1, n2=n2):
                s, j1 = _rint(s, n1)
                s, j2 = _rint(s, n2)
                return s, j1, j2

            def skip(s):
                return s, jnp.uint32(0), jnp.uint32(0)

            st, j1, j2 = lax.cond(valid, do, skip, st)
            tr[p][c] = j1.astype(jnp.int32) + 1
            tr[p + 4][c] = j2.astype(jnp.int32) + 1
            vr[p][c] = valid.astype(jnp.float32)
    targets = jnp.stack([tr[i][c] for i in range(8) for c in range(_K)])
    valid = jnp.stack([jnp.stack(row) for row in vr])
    return targets, valid


_T1 = 6272


def _k1_body(e_ref, m_ref, sums_ref, cnt_ref):
    t = pl.program_id(1)
    e = e_ref[0]
    m = m_ref[0]
    s = lax.dot_general(m, e, (((1,), (1,)), ((), ())),
                        preferred_element_type=jnp.float32)
    c = jnp.sum(m, axis=1, keepdims=True)
    cb = jnp.broadcast_to(c, (8, 128))

    @pl.when(t == 0)
    def _():
        sums_ref[0] = s
        cnt_ref[0] = cb

    @pl.when(t != 0)
    def _():
        sums_ref[0] += s
        cnt_ref[0] += cb


def _k1(embr, mr):
    nt = embr.shape[2] // _T1
    return pl.pallas_call(
        _k1_body,
        grid=(8, nt),
        in_specs=[
            pl.BlockSpec((1, 96, _T1), lambda b, t: (b, 0, t)),
            pl.BlockSpec((1, 8, _T1), lambda b, t: (b, 0, t)),
        ],
        out_specs=[
            pl.BlockSpec((1, 8, 96), lambda b, t: (b, 0, 0)),
            pl.BlockSpec((1, 8, 128), lambda b, t: (b, 0, 0)),
        ],
        out_shape=[
            jax.ShapeDtypeStruct((8, 8, 96), jnp.float32),
            jax.ShapeDtypeStruct((8, 8, 128), jnp.float32),
        ],
    )(embr, mr)


def _k2_body(tr_ref, m_ref, k_ref):
    i = pl.program_id(0)
    x = m_ref[0]
    rows = lax.broadcasted_iota(jnp.int32, (128, 128), 0)
    cols = lax.broadcasted_iota(jnp.int32, (128, 128), 1)
    upper = (rows <= cols).astype(jnp.float32)
    inc = lax.dot_general(x, upper, (((1,), (0,)), ((), ())),
                          preferred_element_type=jnp.float32)
    rowtot = inc[:, 127:128]
    ii = lax.broadcasted_iota(jnp.int32, (392, 392), 0)
    jj = lax.broadcasted_iota(jnp.int32, (392, 392), 1)
    strict = (jj < ii).astype(jnp.float32)
    pre = lax.dot_general(strict, rowtot, (((1,), (0,)), ((), ())),
                          preferred_element_type=jnp.float32)
    cs = pre + inc
    j = (tr_ref[i] - 1).astype(jnp.float32)
    k = jnp.sum(jnp.where(cs <= j, 1.0, 0.0))
    k = jnp.minimum(k, 50175.0).astype(jnp.int32)
    k_ref[0] = jnp.full((1, 128), k, dtype=jnp.int32)


def _k2(targets, m4):
    return pl.pallas_call(
        _k2_body,
        grid_spec=pltpu.PrefetchScalarGridSpec(
            num_scalar_prefetch=1,
            grid=(64,),
            in_specs=[pl.BlockSpec((1, 392, 128), lambda i, tr: (i, 0, 0))],
            out_specs=pl.BlockSpec((1, 1, 128), lambda i, tr: (i, 0, 0)),
        ),
        out_shape=jax.ShapeDtypeStruct((64, 1, 128), jnp.int32),
    )(targets, m4)


def _k3_body(k_ref, e_ref, z_ref):
    i = pl.program_id(0)
    col = k_ref[i] % 512
    e = e_ref[0]
    lane = lax.broadcasted_iota(jnp.int32, (96, 512), 1)
    sel = jnp.where(lane == col, 1.0, 0.0)
    zv = jnp.sum(e * sel, axis=1, keepdims=True)
    z_ref[0] = jnp.broadcast_to(zv, (96, 128))


def _k3(kvec, embr):
    return pl.pallas_call(
        _k3_body,
        grid_spec=pltpu.PrefetchScalarGridSpec(
            num_scalar_prefetch=1,
            grid=(64,),
            in_specs=[
                pl.BlockSpec((1, 96, 512), lambda i, kr: (i // 8, 0, kr[i] // 512)),
            ],
            out_specs=pl.BlockSpec((1, 96, 128), lambda i, kr: (i, 0, 0)),
        ),
        out_shape=jax.ShapeDtypeStruct((64, 96, 128), jnp.float32),
    )(kvec, embr)


def _k4_body(sums_ref, cnt_ref, z_ref, val_ref, out_ref):
    eye_r = lax.broadcasted_iota(jnp.int32, (8, 8), 0)
    eye_c = lax.broadcasted_iota(jnp.int32, (8, 8), 1)
    eye = eye_r == eye_c

    sims = []
    nz = []
    total = jnp.float32(0.0)
    count = jnp.float32(0.0)
    for img in range(8):
        cnt_row = cnt_ref[img:img + 1, :]
        cnt_col = jnp.transpose(cnt_row)
        mean = sums_ref[img] / jnp.maximum(cnt_col, 1.0)
        z = z_ref[img]
        nm = jnp.sqrt(jnp.sum(mean * mean, axis=1, keepdims=True))
        nzv = jnp.sqrt(jnp.sum(z * z, axis=1, keepdims=True))
        d = lax.dot_general(z, mean, (((1,), (1,)), ((), ())),
                            preferred_element_type=jnp.float32)
        den = jnp.maximum(nzv * jnp.transpose(nm), 1e-8)
        sims.append(d / den / _TEMP)
        nz.append(cnt_row > 0.0)

    for p in range(4):
        s1, s2 = sims[p], sims[p + 4]
        nz1, nz2 = nz[p], nz[p + 4]

        def _loss(s, nzrow):
            pos = jnp.sum(jnp.where(eye, s, 0.0), axis=1, keepdims=True)
            vals = jnp.where(eye | nzrow, s, _NEG_INF)
            mx = jnp.max(vals, axis=1, keepdims=True)
            den_main = mx + jnp.log(jnp.sum(jnp.exp(vals - mx), axis=1, keepdims=True))
            has = jnp.sum(jnp.where((~eye) & nzrow, 1.0, 0.0), axis=1, keepdims=True) > 0.0
            den = jnp.where(has, den_main, pos + _C0)
            return den - pos

        l1 = _loss(s1, nz1)
        l2 = _loss(s2, nz2)
        v = val_ref[p:p + 1, :]
        contrib = jnp.transpose(v) * 0.5 * (l1 + l2)
        total = total + jnp.sum(contrib)
        count = count + jnp.sum(v)

    res = jnp.where(count > 0.0, total / jnp.maximum(count, 1.0), 0.0)
    out_ref[...] = jnp.full((8, 128), res, dtype=jnp.float32)


def _k4(sums, counts, z, valid):
    return pl.pallas_call(
        _k4_body,
        out_shape=jax.ShapeDtypeStruct((8, 128), jnp.float32),
    )(sums, counts, z, valid)


def kernel(embeddings, masks_onehot):
    B, E, H, W = embeddings.shape
    HW = H * W
    mf = masks_onehot.astype(jnp.float32)
    embr = embeddings.reshape(B, E, HW)
    mr = mf.reshape(B, _K, HW)
    m4 = mf.reshape(B * _K, HW // 128, 128)

    sums, cnts = _k1(embr, mr)
    counts_f = cnts[:, :, 0]
    counts_i = counts_f.astype(jnp.int32)

    targets, valid = _draw_targets(counts_i)

    kk = _k2(targets, m4)
    kvec = kk[:, 0, 0]

    zfull = _k3(kvec, embr)
    z = zfull[:, :, 0].reshape(8, _K, E)

    out = _k4(sums, counts_f, z, valid)
    return out[0, 0]
